# baseline (speedup 1.0000x reference)
import jax
import jax.numpy as jnp
from jax import lax
from jax.experimental import pallas as pl
from jax.experimental.pallas import tpu as pltpu
from jax.experimental.pallas import tpu_sc as plsc

_B = 16
_M = 8192
_H = 512
_W = 512
_NC = 2
_HALVES = 2
_HROWS = _H // _HALVES
_SUBS = 4
_ROWS = _HROWS // _SUBS
_REGION = _ROWS * _W

_LANES = 16
_ZUNROLL = 16
_CUNROLL = 2


def _zero(buf):
  def body(i, _):
    base = i * (_LANES * _ZUNROLL)
    for u in range(_ZUNROLL):
      buf[pl.ds(base + u * _LANES, _LANES)] = jnp.zeros(
          (_LANES,), jnp.float32)
    return 0

  lax.fori_loop(0, _REGION // (_LANES * _ZUNROLL), body, 0)


def _tile_body(idx_hbm, nv_hbm, val_hbm, out_hbm,
               idx_v, val_v, nv_v, mlist, dense0, dense1,
               sem_i, sem_v, sem_n, sem_o0, sem_o1):
  wid = lax.axis_index("s") * _NC + lax.axis_index("c")
  b = wid // _HALVES
  half = wid % _HALVES
  h_lo = half * _HROWS

  cp_i = pltpu.async_copy(idx_hbm.at[b], idx_v, sem_i)
  cp_v = pltpu.async_copy(val_hbm.at[b], val_v, sem_v)
  cp_n = pltpu.async_copy(nv_hbm, nv_v, sem_n)

  _zero(dense0)
  _zero(dense1)

  cp_n.wait()
  cp_i.wait()
  cp_v.wait()

  lane = lax.iota(jnp.int32, _LANES)
  b_splat = jnp.full((_LANES,), b, jnp.int32)
  nv_splat = plsc.load_gather(nv_v, [b_splat])
  n_chunks = (jnp.max(nv_splat) + (_LANES - 1)) // _LANES

  def compact_body(i, cnt):
    mm = i * _LANES + lane
    off = ((mm >> 7) << 8) + (mm & 127)
    r = plsc.load_gather(idx_v, [off])
    rel = r - h_lo
    msk = (mm < nv_splat) & (rel >= 0) & (rel < _HROWS)
    pos = plsc.cumsum(jnp.where(msk, 1, 0))
    plsc.store_scatter(mlist, [jnp.where(msk, cnt + pos - 1, 0)], mm,
                       mask=msk)
    return cnt + plsc.all_reduce_population_count(msk)

  cnt = lax.fori_loop(0, n_chunks, compact_body,
                      jnp.zeros((_LANES,), jnp.int32))
  n_half = jnp.max(cnt)
  n_blocks = (n_half + (_LANES * _CUNROLL - 1)) // (_LANES * _CUNROLL)

  bufs = (dense0, dense1)
  sems = (sem_o0, sem_o1)
  out_cp = [None, None]

  for sub in range(_SUBS):
    buf = bufs[sub % 2]
    sem = sems[sub % 2]
    r_lo = h_lo + sub * _ROWS

    if sub >= 2:
      out_cp[sub % 2].wait()
      _zero(buf)

    def scatter_body(i, _, buf=buf, r_lo=r_lo):
      for u in range(_CUNROLL):
        k0 = (i * _CUNROLL + u) * _LANES
        kk = k0 + lane
        inb = kk < cnt
        mm = jnp.where(inb, mlist[pl.ds(k0, _LANES)], 0)
        off = ((mm >> 7) << 8) + (mm & 127)
        r = plsc.load_gather(idx_v, [off])
        c = plsc.load_gather(idx_v, [off + 128])
        v = plsc.load_gather(val_v, [mm])
        rel = r - r_lo
        msk = inb & (rel >= 0) & (rel < _ROWS)
        relc = jnp.where(msk, rel, 0)
        loc = ((relc >> 3) << 12) + ((c >> 7) << 10) + ((relc & 7) << 7) + (
            c & 127)
        plsc.store_scatter(buf, [loc], v, mask=msk)
      return 0

    lax.fori_loop(0, n_blocks, scatter_body, 0)

    out_cp[sub % 2] = pltpu.async_copy(
        buf, out_hbm.at[b, pl.ds(r_lo * _W, _REGION)], sem)

  out_cp[0].wait()
  out_cp[1].wait()


@jax.jit
def _launch(idx_flat, num_valid, vals):
  mesh = plsc.VectorSubcoreMesh(core_axis_name="c", subcore_axis_name="s")
  f = pl.kernel(
      _tile_body,
      out_type=jax.ShapeDtypeStruct((_B, _H * _W), jnp.float32),
      mesh=mesh,
      compiler_params=pltpu.CompilerParams(
          needs_layout_passes=False, use_tc_tiling_on_sc=False,
          disable_bounds_checks=True, disable_semaphore_checks=True,
          skip_device_barrier=True),
      scratch_types=[
          pltpu.VMEM((_M * 2,), jnp.int32),
          pltpu.VMEM((_M,), jnp.float32),
          pltpu.VMEM((_LANES,), jnp.int32),
          pltpu.VMEM((_M,), jnp.int32),
          pltpu.VMEM((_REGION,), jnp.float32),
          pltpu.VMEM((_REGION,), jnp.float32),
          pltpu.SemaphoreType.DMA,
          pltpu.SemaphoreType.DMA,
          pltpu.SemaphoreType.DMA,
          pltpu.SemaphoreType.DMA,
          pltpu.SemaphoreType.DMA,
      ],
  )
  return f(idx_flat, num_valid, vals)


def kernel(indices, num_valid_coordinates, padded_features):
  idx_flat = indices.reshape(_B, _M // 128, 128, 2)
  idx_flat = idx_flat.transpose(0, 1, 3, 2).reshape(_B, 2 * _M)
  vals = padded_features.reshape(_B, _M)
  out = _launch(idx_flat, num_valid_coordinates, vals)
  out = out.reshape(_B, _H // 8, _W // 128, 8, 128)
  out = out.transpose(0, 1, 3, 2, 4)
  return out.reshape(_B, _H, _W)


# speedup vs baseline: 1.2629x; 1.2629x over previous
import jax
import jax.numpy as jnp
from jax import lax
from jax.experimental import pallas as pl
from jax.experimental.pallas import tpu as pltpu
from jax.experimental.pallas import tpu_sc as plsc

_B = 16
_M = 8192
_H = 512
_W = 512
_NC = 2
_HALVES = 2
_SUBS = 2
_ROWS = _H // (_HALVES * _SUBS)
_REGION = _ROWS * _W

_LANES = 16
_ZUNROLL = 16
_CUNROLL = 2


def _tile_body(idx_hbm, nv_hbm, val_hbm, out_hbm,
               idx_v, val_v, nv_v, dense_v, sem_i, sem_v, sem_n):
  wid = lax.axis_index("s") * _NC + lax.axis_index("c")
  b = wid // _HALVES
  half = wid % _HALVES

  cp_i = pltpu.async_copy(idx_hbm.at[b], idx_v, sem_i)
  cp_v = pltpu.async_copy(val_hbm.at[b], val_v, sem_v)
  cp_n = pltpu.async_copy(nv_hbm, nv_v, sem_n)

  def zero_body(i, _):
    base = i * (_LANES * _ZUNROLL)
    for u in range(_ZUNROLL):
      dense_v[pl.ds(base + u * _LANES, _LANES)] = jnp.zeros(
          (_LANES,), jnp.float32)
    return 0

  lax.fori_loop(0, _REGION // (_LANES * _ZUNROLL), zero_body, 0)

  cp_n.wait()
  cp_i.wait()
  cp_v.wait()

  lane = lax.iota(jnp.int32, _LANES)
  b_splat = jnp.full((_LANES,), b, jnp.int32)
  nv_splat = plsc.load_gather(nv_v, [b_splat])
  n_blocks = (jnp.max(nv_splat) + (_LANES * _CUNROLL - 1)) // (
      _LANES * _CUNROLL)

  for sub in range(_SUBS):
    r_lo = (half * _SUBS + sub) * _ROWS

    if sub > 0:
      lax.fori_loop(0, _REGION // (_LANES * _ZUNROLL), zero_body, 0)

    def scatter_body(i, _, r_lo=r_lo):
      for u in range(_CUNROLL):
        m0 = (i * _CUNROLL + u) * _LANES
        mm = m0 + lane
        off = ((mm >> 7) << 8) + (mm & 127)
        r = plsc.load_gather(idx_v, [off])
        c = plsc.load_gather(idx_v, [off + 128])
        v = val_v[pl.ds(m0, _LANES)]
        rel = r - r_lo
        msk = (mm < nv_splat) & (rel >= 0) & (rel < _ROWS)
        relc = jnp.where(msk, rel, 0)
        loc = ((relc >> 3) << 12) + ((c >> 7) << 10) + ((relc & 7) << 7) + (
            c & 127)
        plsc.store_scatter(dense_v, [loc], v, mask=msk)
      return 0

    lax.fori_loop(0, n_blocks, scatter_body, 0)

    pltpu.sync_copy(dense_v, out_hbm.at[b, pl.ds(r_lo * _W, _REGION)])


@jax.jit
def _launch(idx_flat, num_valid, vals):
  mesh = plsc.VectorSubcoreMesh(core_axis_name="c", subcore_axis_name="s")
  f = pl.kernel(
      _tile_body,
      out_type=jax.ShapeDtypeStruct((_B, _H * _W), jnp.float32),
      mesh=mesh,
      compiler_params=pltpu.CompilerParams(
          needs_layout_passes=False, use_tc_tiling_on_sc=False,
          disable_bounds_checks=True, disable_semaphore_checks=True,
          skip_device_barrier=True),
      scratch_types=[
          pltpu.VMEM((_M * 2,), jnp.int32),
          pltpu.VMEM((_M,), jnp.float32),
          pltpu.VMEM((_LANES,), jnp.int32),
          pltpu.VMEM((_REGION,), jnp.float32),
          pltpu.SemaphoreType.DMA,
          pltpu.SemaphoreType.DMA,
          pltpu.SemaphoreType.DMA,
      ],
  )
  return f(idx_flat, num_valid, vals)


def kernel(indices, num_valid_coordinates, padded_features):
  idx_flat = indices.reshape(_B, _M // 128, 128, 2)
  idx_flat = idx_flat.transpose(0, 1, 3, 2).reshape(_B, 2 * _M)
  vals = padded_features.reshape(_B, _M)
  out = _launch(idx_flat, num_valid_coordinates, vals)
  out = out.reshape(_B, _H // 8, _W // 128, 8, 128)
  out = out.transpose(0, 1, 3, 2, 4)
  return out.reshape(_B, _H, _W)
